# baseline (speedup 1.0000x reference)
import functools

import jax
import jax.numpy as jnp
from jax import lax
from jax.experimental import pallas as pl
from jax.experimental.pallas import tpu as pltpu
from jax.experimental.pallas import tpu_sc as plsc

NC = 2
NS = 16
NW = NC * NS
CHUNK = 100
NBUF = 4


def _sc_gather_maxpool(x2, emb, B, S, D):
    bpw = B // NW
    cpr = S // CHUNK
    cpw = bpw * cpr
    nlc = D // 16

    mesh = plsc.VectorSubcoreMesh(
        core_axis_name="c", subcore_axis_name="s",
        num_cores=NC, num_subcores=NS)

    @functools.partial(
        pl.kernel,
        out_type=jax.ShapeDtypeStruct((B, D), jnp.float32),
        mesh=mesh,
        scratch_types=[
            pltpu.VMEM((cpw, CHUNK), jnp.int32),
            pltpu.VMEM((NBUF, S, D), jnp.float32),
            pltpu.VMEM((bpw, D), jnp.float32),
        ] + [pltpu.SemaphoreType.DMA] * NBUF,
        compiler_params=pltpu.CompilerParams(use_tc_tiling_on_sc=False),
    )
    def pool_kernel(x_hbm, emb_hbm, out_hbm, idx_v, rows_v, out_v, *sems):
        wid = lax.axis_index("s") * NC + lax.axis_index("c")
        pltpu.sync_copy(x_hbm.at[pl.ds(wid * cpw, cpw)], idx_v)

        def start_row(row, buf):
            for h in range(cpr):
                pltpu.async_copy(
                    emb_hbm.at[idx_v.at[row * cpr + h]],
                    rows_v.at[buf, pl.ds(h * CHUNK, CHUNK)],
                    sems[buf])

        def wait_row(buf):
            for h in range(cpr):
                pltpu.make_async_copy(
                    emb_hbm.at[idx_v.at[0]],
                    rows_v.at[buf, pl.ds(h * CHUNK, CHUNK)],
                    sems[buf]).wait()

        def reduce_row(row, buf):
            def body(j, accs):
                return tuple(
                    jnp.maximum(accs[c], rows_v[buf, j, pl.ds(c * 16, 16)])
                    for c in range(nlc))
            inits = tuple(rows_v[buf, 0, pl.ds(c * 16, 16)] for c in range(nlc))
            accs = lax.fori_loop(1, S, body, inits, unroll=4)
            for c in range(nlc):
                out_v[row, pl.ds(c * 16, 16)] = accs[c]

        for b in range(NBUF):
            start_row(b, b)

        def loop_body(g, carry):
            for b in range(NBUF):
                row = g * NBUF + b
                wait_row(b)
                reduce_row(row, b)

                @pl.when(row + NBUF < bpw)
                def _():
                    start_row(row + NBUF, b)
            return carry

        lax.fori_loop(0, bpw // NBUF, loop_body, 0)
        pltpu.sync_copy(out_v, out_hbm.at[pl.ds(wid * bpw, bpw)])

    return pool_kernel(x2, emb)


def _tc_linear(pooled, W, b2):
    def mm_body(p_ref, w_ref, b_ref, o_ref):
        o_ref[...] = lax.dot_general(
            p_ref[...], w_ref[...],
            (((1,), (1,)), ((), ())),
            preferred_element_type=jnp.float32) + b_ref[...]

    return pl.pallas_call(
        mm_body,
        out_shape=jax.ShapeDtypeStruct((pooled.shape[0], W.shape[0]),
                                       jnp.float32),
    )(pooled, W, b2)


@jax.jit
def kernel(x, emb, W, b):
    B, S = x.shape
    V, D = emb.shape
    x2 = x.astype(jnp.int32).reshape(B * S // CHUNK, CHUNK)
    pooled = _sc_gather_maxpool(x2, emb, B, S, D)
    return _tc_linear(pooled, W, b.reshape(1, -1))
